# baseline (speedup 1.0000x reference)
import functools

import jax
import jax.numpy as jnp
from jax import lax
from jax.experimental import pallas as pl
from jax.experimental.pallas import tpu as pltpu
from jax.experimental.pallas import tpu_sc as plsc

B = 16384
EMB = 50

NC = 2
NS = 16
NW = NC * NS
BPW = B // NW
CHUNK = 128
NCHUNK = BPW // CHUNK

_sc_mesh = plsc.VectorSubcoreMesh(core_axis_name="c", subcore_axis_name="s")


def _sc_gather_body(u_tab, m_tab, uidx_hbm, midx_hbm, u_out, m_out,
                    idx_u, idx_m, rows_u, rows_m, sem):
    wid = lax.axis_index("s") * NC + lax.axis_index("c")
    pltpu.sync_copy(uidx_hbm.at[wid], idx_u)
    pltpu.sync_copy(midx_hbm.at[wid], idx_m)
    copies = []
    for j in range(NCHUNK):
        copies.append(pltpu.async_copy(
            u_tab.at[idx_u.at[j]], rows_u.at[pl.ds(j * CHUNK, CHUNK)], sem))
        copies.append(pltpu.async_copy(
            m_tab.at[idx_m.at[j]], rows_m.at[pl.ds(j * CHUNK, CHUNK)], sem))
    for c in copies:
        c.wait()
    base = wid * BPW
    pltpu.sync_copy(rows_u, u_out.at[pl.ds(base, BPW)])
    pltpu.sync_copy(rows_m, m_out.at[pl.ds(base, BPW)])


def _make_sc_gather(interpret=False):
    return functools.partial(
        pl.kernel,
        mesh=_sc_mesh,
        interpret=interpret,
        compiler_params=pltpu.CompilerParams(use_tc_tiling_on_sc=False),
        out_type=[
            jax.ShapeDtypeStruct((B, EMB), jnp.float32),
            jax.ShapeDtypeStruct((B, EMB), jnp.float32),
        ],
        scratch_types=[
            pltpu.VMEM((NCHUNK, CHUNK), jnp.int32),
            pltpu.VMEM((NCHUNK, CHUNK), jnp.int32),
            pltpu.VMEM((BPW, EMB), jnp.float32),
            pltpu.VMEM((BPW, EMB), jnp.float32),
            pltpu.SemaphoreType.DMA,
        ],
    )(_sc_gather_body)


_sc_gather = _make_sc_gather()


_ROWS_BLK = 2048


def _mlp_body(u_ref, m_ref, w1a_ref, w1b_ref, b1_ref, w2_ref, b2_ref, o_ref):
    h = (jnp.dot(u_ref[...], w1a_ref[...], preferred_element_type=jnp.float32)
         + jnp.dot(m_ref[...], w1b_ref[...], preferred_element_type=jnp.float32)
         + b1_ref[...])
    h = jnp.maximum(h, 0.0)
    o_ref[...] = jnp.dot(h, w2_ref[...], preferred_element_type=jnp.float32) + b2_ref[...]


def _mlp(u, m, w1a, w1b, b1, w2, b2):
    grid = (B // _ROWS_BLK,)
    return pl.pallas_call(
        _mlp_body,
        grid=grid,
        in_specs=[
            pl.BlockSpec((_ROWS_BLK, EMB), lambda i: (i, 0)),
            pl.BlockSpec((_ROWS_BLK, EMB), lambda i: (i, 0)),
            pl.BlockSpec((EMB, 10), lambda i: (0, 0)),
            pl.BlockSpec((EMB, 10), lambda i: (0, 0)),
            pl.BlockSpec((1, 10), lambda i: (0, 0)),
            pl.BlockSpec((10, 1), lambda i: (0, 0)),
            pl.BlockSpec((1, 1), lambda i: (0, 0)),
        ],
        out_specs=pl.BlockSpec((_ROWS_BLK, 1), lambda i: (i, 0)),
        out_shape=jax.ShapeDtypeStruct((B, 1), jnp.float32),
    )(u, m, w1a, w1b, b1, w2, b2)


def kernel(users, movies, user_table, movie_table, W1, b1, W2, b2):
    uidx = users.astype(jnp.int32).reshape(NW, NCHUNK, CHUNK)
    midx = movies.astype(jnp.int32).reshape(NW, NCHUNK, CHUNK)
    u, m = _sc_gather(user_table, movie_table, uidx, midx)
    h = jnp.maximum(u @ W1[:EMB] + m @ W1[EMB:] + b1, 0.0)
    return h @ W2 + b2


# speedup vs baseline: 2.3630x; 2.3630x over previous
import functools

import jax
import jax.numpy as jnp
from jax import lax
from jax.experimental import pallas as pl
from jax.experimental.pallas import tpu as pltpu
from jax.experimental.pallas import tpu_sc as plsc

B = 16384
EMB = 50
HID = 10
HIDP = 16
N_U = 1000000
N_M = 100000
CHUNK_U = 8064
NFULL_U = 124
N_U_MAIN = CHUNK_U * NFULL_U
TAIL_U = N_U - N_U_MAIN
CHUNK_M = 8192
LAST_M = 1664
N_M_MAIN = 12 * CHUNK_M + LAST_M
TAIL_M = N_M - N_M_MAIN

NC = 2
NS = 16
NW = NC * NS
BPW = B // NW
NCHUNK = BPW // 128

_B1_ROW = 0
_W2_ROW = 10
_B2_ROW = 20
_WTAB_ROWS = 24


def _proj_body(nchunks, chunk, last, tab_ref, w_ref, out_ref,
               xb, ob, semx, semo):
    i = pl.program_id(0)
    slot = i % 2
    ragged = last != chunk

    def in_copy(ci, sl, size):
        return pltpu.make_async_copy(
            tab_ref.at[:, pl.ds(ci * chunk, size)],
            xb.at[sl, :, pl.ds(0, size)], semx.at[sl])

    def out_copy(ci, sl, size):
        return pltpu.make_async_copy(
            ob.at[sl, :, pl.ds(0, size)],
            out_ref.at[:, pl.ds(ci * chunk, size)], semo.at[sl])

    def start_in(ci, sl):
        if ragged:
            @pl.when(ci == nchunks - 1)
            def _():
                in_copy(ci, sl, last).start()

            @pl.when(ci < nchunks - 1)
            def _():
                in_copy(ci, sl, chunk).start()
        else:
            in_copy(ci, sl, chunk).start()

    def wait_in(ci, sl):
        if ragged:
            @pl.when(ci == nchunks - 1)
            def _():
                in_copy(ci, sl, last).wait()

            @pl.when(ci < nchunks - 1)
            def _():
                in_copy(ci, sl, chunk).wait()
        else:
            in_copy(ci, sl, chunk).wait()

    def start_out(ci, sl):
        if ragged:
            @pl.when(ci == nchunks - 1)
            def _():
                out_copy(ci, sl, last).start()

            @pl.when(ci < nchunks - 1)
            def _():
                out_copy(ci, sl, chunk).start()
        else:
            out_copy(ci, sl, chunk).start()

    def wait_out(ci, sl):
        if ragged:
            @pl.when(ci == nchunks - 1)
            def _():
                out_copy(ci, sl, last).wait()

            @pl.when(ci < nchunks - 1)
            def _():
                out_copy(ci, sl, chunk).wait()
        else:
            out_copy(ci, sl, chunk).wait()

    @pl.when(i == 0)
    def _():
        start_in(0, 0)

    @pl.when((i + 1) < nchunks)
    def _():
        start_in(i + 1, 1 - slot)

    wait_in(i, slot)
    h = lax.dot_general(w_ref[...], xb[slot], (((1,), (0,)), ((), ())),
                        precision=lax.Precision.HIGHEST,
                        preferred_element_type=jnp.float32)

    @pl.when(i >= 2)
    def _():
        wait_out(i - 2, slot)

    ob[slot] = h
    start_out(i, slot)

    @pl.when(i == nchunks - 1)
    def _():
        wait_out(i, slot)

    @pl.when((i == nchunks - 1) & (i >= 1))
    def _():
        wait_out(i - 1, 1 - slot)


def _project(tabT, w16, n, chunk, last):
    nchunks = (n - last) // chunk + 1
    return pl.pallas_call(
        functools.partial(_proj_body, nchunks, chunk, last),
        grid=(nchunks,),
        in_specs=[
            pl.BlockSpec(memory_space=pl.ANY),
            pl.BlockSpec((HIDP, EMB), lambda i: (0, 0)),
        ],
        out_specs=pl.BlockSpec(memory_space=pl.ANY),
        out_shape=jax.ShapeDtypeStruct((HIDP, n), jnp.float32),
        scratch_shapes=[
            pltpu.VMEM((2, EMB, chunk), jnp.float32),
            pltpu.VMEM((2, HIDP, chunk), jnp.float32),
            pltpu.SemaphoreType.DMA((2,)),
            pltpu.SemaphoreType.DMA((2,)),
        ],
    )(tabT, w16)


def _tail_body(w_ref, t_ref, o_ref):
    o_ref[...] = lax.dot_general(w_ref[...], t_ref[...],
                                 (((1,), (0,)), ((), ())),
                                 precision=lax.Precision.HIGHEST,
                                 preferred_element_type=jnp.float32)


def _tail_proj(tailT, w16T, tail_n):
    return pl.pallas_call(
        _tail_body,
        out_shape=jax.ShapeDtypeStruct((HIDP, tail_n), jnp.float32),
    )(w16T, tailT)


_sc_mesh = plsc.VectorSubcoreMesh(core_axis_name="c", subcore_axis_name="s")


def _sc_body(pu8, pm8, uidx8_hbm, uoff_hbm, midx8_hbm, moff_hbm, wtab_hbm,
             out_hbm, i8u0, i8u1, i8u2, i8u3, i8m0, i8m1, i8m2, i8m3,
             off_u, off_m, urows0, urows1, mrows0, mrows1,
             wv, outv, semu0, semu1, semm0, semm1):
    wid = lax.axis_index("s") * NC + lax.axis_index("c")
    i8u = (i8u0, i8u1, i8u2, i8u3)
    i8m = (i8m0, i8m1, i8m2, i8m3)
    for j in range(NCHUNK):
        pltpu.sync_copy(uidx8_hbm.at[wid, j], i8u[j])
        pltpu.sync_copy(midx8_hbm.at[wid, j], i8m[j])
    pltpu.sync_copy(uoff_hbm.at[wid], off_u)
    pltpu.sync_copy(moff_hbm.at[wid], off_m)
    pltpu.sync_copy(wtab_hbm, wv)

    semu = (semu0, semu1)
    semm = (semm0, semm1)
    urows = (urows0, urows1)
    mrows = (mrows0, mrows1)

    def start(j):
        sl = j % 2
        return (pltpu.async_copy(pu8.at[i8u[j]], urows[sl], semu[sl]),
                pltpu.async_copy(pm8.at[i8m[j]], mrows[sl], semm[sl]))

    lanes = lax.iota(jnp.int32, 16)
    inflight = {0: start(0), 1: start(1)}
    for j in range(NCHUNK):
        sl = j % 2
        cu, cm = inflight.pop(j)
        cu.wait()
        cm.wait()
        for g in range(8):
            gsl = pl.ds(g * 16, 16)
            offu = off_u[j, gsl]
            offm = off_m[j, gsl]
            rows = lanes + g * 16
            acc = wv[_B2_ROW, :]
            for jh in range(HID):
                vu = plsc.load_gather(urows[sl], [rows, offu + jh])
                vm = plsc.load_gather(mrows[sl], [rows, offm + jh])
                h = jnp.maximum(vu + vm + wv[_B1_ROW + jh, :], 0.0)
                acc = acc + h * wv[_W2_ROW + jh, :]
            outv[pl.ds(j * 128 + g * 16, 16)] = acc
        if j + 2 < NCHUNK:
            inflight[j + 2] = start(j + 2)

    plsc.subcore_barrier()
    pltpu.sync_copy(outv, out_hbm.at[pl.ds(wid * BPW, BPW)])


_sc_gather_mlp = functools.partial(
    pl.kernel,
    mesh=_sc_mesh,
    compiler_params=pltpu.CompilerParams(needs_layout_passes=False),
    out_type=jax.ShapeDtypeStruct((B,), jnp.float32),
    scratch_types=[
        pltpu.VMEM((128,), jnp.int32),
        pltpu.VMEM((128,), jnp.int32),
        pltpu.VMEM((128,), jnp.int32),
        pltpu.VMEM((128,), jnp.int32),
        pltpu.VMEM((128,), jnp.int32),
        pltpu.VMEM((128,), jnp.int32),
        pltpu.VMEM((128,), jnp.int32),
        pltpu.VMEM((128,), jnp.int32),
        pltpu.VMEM((NCHUNK, 128), jnp.int32),
        pltpu.VMEM((NCHUNK, 128), jnp.int32),
        pltpu.VMEM((128, 128), jnp.float32),
        pltpu.VMEM((128, 128), jnp.float32),
        pltpu.VMEM((128, 128), jnp.float32),
        pltpu.VMEM((128, 128), jnp.float32),
        pltpu.VMEM((_WTAB_ROWS, 16), jnp.float32),
        pltpu.VMEM((BPW,), jnp.float32),
        pltpu.SemaphoreType.DMA,
        pltpu.SemaphoreType.DMA,
        pltpu.SemaphoreType.DMA,
        pltpu.SemaphoreType.DMA,
    ],
)(_sc_body)


def kernel(users, movies, user_table, movie_table, W1, b1, W2, b2):
    w1aT = jnp.pad(W1[:EMB], ((0, 0), (0, HIDP - HID))).T
    w1bT = jnp.pad(W1[EMB:], ((0, 0), (0, HIDP - HID))).T
    tabTu = user_table.T
    tabTm = movie_table.T
    put = _project(tabTu, w1aT, N_U_MAIN, CHUNK_U, CHUNK_U)
    tailpu = _tail_proj(lax.slice(tabTu, (0, N_U_MAIN), (EMB, N_U)),
                        w1aT, TAIL_U)
    pmt = _project(tabTm, w1bT, N_M_MAIN, CHUNK_M, LAST_M)
    tailpm = _tail_proj(lax.slice(tabTm, (0, N_M_MAIN), (EMB, N_M)),
                        w1bT, TAIL_M)
    pu8 = jnp.concatenate([put, tailpu], axis=1).T.reshape(N_U // 8, 128)
    pm8 = jnp.concatenate([pmt, tailpm], axis=1).T.reshape(N_M // 8, 128)

    ones = jnp.ones((16,), jnp.float32)
    wtab = jnp.zeros((_WTAB_ROWS, 16), jnp.float32)
    wtab = wtab.at[_B1_ROW:_B1_ROW + HID].set(b1[:, None] * ones)
    wtab = wtab.at[_W2_ROW:_W2_ROW + HID].set(W2[:, 0][:, None] * ones)
    wtab = wtab.at[_B2_ROW].set(b2[0] * ones)

    ui = users.astype(jnp.int32)
    mi = movies.astype(jnp.int32)
    out = _sc_gather_mlp(
        pu8, pm8,
        (ui >> 3).reshape(NW, NCHUNK, 128),
        ((ui & 7) << 4).reshape(NW, NCHUNK, 128),
        (mi >> 3).reshape(NW, NCHUNK, 128),
        ((mi & 7) << 4).reshape(NW, NCHUNK, 128),
        wtab)
    return out.reshape(B, 1)


# speedup vs baseline: 2.7763x; 1.1749x over previous
import functools

import jax
import jax.numpy as jnp
from jax import lax
from jax.experimental import pallas as pl
from jax.experimental.pallas import tpu as pltpu
from jax.experimental.pallas import tpu_sc as plsc

B = 16384
EMB = 50
HID = 10
HIDP = 16
N_U = 1000000
N_M = 100000
CHUNK_U = 8064
NFULL_U = 124
N_U_MAIN = CHUNK_U * NFULL_U
TAIL_U = N_U - N_U_MAIN
CHUNK_M = 8192
LAST_M = 1664
N_M_MAIN = 12 * CHUNK_M + LAST_M
TAIL_M = N_M - N_M_MAIN

NC = 2
NS = 16
NW = NC * NS
BPW = B // NW
NCHUNK = BPW // 128

_B1_ROW = 0
_W2_ROW = 10
_B2_ROW = 20
_WTAB_ROWS = 24


def _proj_body(nchunks, chunk, last, tab_ref, w_ref, out_ref,
               xb, ob, semx, semo):
    i = pl.program_id(0)
    slot = i % 2
    ragged = last != chunk

    def in_copy(ci, sl, size):
        return pltpu.make_async_copy(
            tab_ref.at[:, pl.ds(ci * chunk, size)],
            xb.at[sl, :, pl.ds(0, size)], semx.at[sl])

    def out_copy(ci, sl, size):
        return pltpu.make_async_copy(
            ob.at[sl, :, pl.ds(0, size)],
            out_ref.at[:, pl.ds(ci * chunk, size)], semo.at[sl])

    def start_in(ci, sl):
        if ragged:
            @pl.when(ci == nchunks - 1)
            def _():
                in_copy(ci, sl, last).start()

            @pl.when(ci < nchunks - 1)
            def _():
                in_copy(ci, sl, chunk).start()
        else:
            in_copy(ci, sl, chunk).start()

    def wait_in(ci, sl):
        if ragged:
            @pl.when(ci == nchunks - 1)
            def _():
                in_copy(ci, sl, last).wait()

            @pl.when(ci < nchunks - 1)
            def _():
                in_copy(ci, sl, chunk).wait()
        else:
            in_copy(ci, sl, chunk).wait()

    def start_out(ci, sl):
        if ragged:
            @pl.when(ci == nchunks - 1)
            def _():
                out_copy(ci, sl, last).start()

            @pl.when(ci < nchunks - 1)
            def _():
                out_copy(ci, sl, chunk).start()
        else:
            out_copy(ci, sl, chunk).start()

    def wait_out(ci, sl):
        if ragged:
            @pl.when(ci == nchunks - 1)
            def _():
                out_copy(ci, sl, last).wait()

            @pl.when(ci < nchunks - 1)
            def _():
                out_copy(ci, sl, chunk).wait()
        else:
            out_copy(ci, sl, chunk).wait()

    @pl.when(i == 0)
    def _():
        start_in(0, 0)

    @pl.when((i + 1) < nchunks)
    def _():
        start_in(i + 1, 1 - slot)

    wait_in(i, slot)
    h = lax.dot_general(w_ref[...], xb[slot], (((1,), (0,)), ((), ())),
                        precision=lax.Precision.HIGHEST,
                        preferred_element_type=jnp.float32)
    @pl.when(i >= 2)
    def _():
        wait_out(i - 2, slot)

    ob[slot] = h
    start_out(i, slot)

    @pl.when(i == nchunks - 1)
    def _():
        wait_out(i, slot)

    @pl.when((i == nchunks - 1) & (i >= 1))
    def _():
        wait_out(i - 1, 1 - slot)


def _project(tabT, w16, n, chunk, last):
    nchunks = (n - last) // chunk + 1
    return pl.pallas_call(
        functools.partial(_proj_body, nchunks, chunk, last),
        grid=(nchunks,),
        in_specs=[
            pl.BlockSpec(memory_space=pl.ANY),
            pl.BlockSpec((HIDP, EMB), lambda i: (0, 0)),
        ],
        out_specs=pl.BlockSpec(memory_space=pl.ANY),
        out_shape=jax.ShapeDtypeStruct((HIDP, n), jnp.float32),
        scratch_shapes=[
            pltpu.VMEM((2, EMB, chunk), jnp.float32),
            pltpu.VMEM((2, HIDP, chunk), jnp.float32),
            pltpu.SemaphoreType.DMA((2,)),
            pltpu.SemaphoreType.DMA((2,)),
        ],
    )(tabT, w16)


def _tail_body(w_ref, t_ref, o_ref):
    o_ref[...] = lax.dot_general(w_ref[...], t_ref[...],
                                 (((1,), (0,)), ((), ())),
                                 precision=lax.Precision.HIGHEST,
                                 preferred_element_type=jnp.float32)


def _tail_proj(tailT, w16T, tail_n):
    return pl.pallas_call(
        _tail_body,
        out_shape=jax.ShapeDtypeStruct((HIDP, tail_n), jnp.float32),
    )(w16T, tailT)


_sc_mesh = plsc.VectorSubcoreMesh(core_axis_name="c", subcore_axis_name="s")


def _sc_body(pu8, pm8, tailu_hbm, tailm_hbm, uidx8_hbm, uoff_hbm, utf_hbm,
             midx8_hbm, moff_hbm, mtf_hbm, wtab_hbm,
             out_hbm, i8u0, i8u1, i8u2, i8u3, i8m0, i8m1, i8m2, i8m3,
             off_u, off_m, tf_u, tf_m, tailu_v, tailm_v,
             urows0, urows1, mrows0, mrows1,
             wv, outv, semu0, semu1, semm0, semm1):
    wid = lax.axis_index("s") * NC + lax.axis_index("c")
    i8u = (i8u0, i8u1, i8u2, i8u3)
    i8m = (i8m0, i8m1, i8m2, i8m3)
    for j in range(NCHUNK):
        pltpu.sync_copy(uidx8_hbm.at[wid, j], i8u[j])
        pltpu.sync_copy(midx8_hbm.at[wid, j], i8m[j])
    pltpu.sync_copy(uoff_hbm.at[wid], off_u)
    pltpu.sync_copy(moff_hbm.at[wid], off_m)
    pltpu.sync_copy(utf_hbm.at[wid], tf_u)
    pltpu.sync_copy(mtf_hbm.at[wid], tf_m)
    pltpu.sync_copy(tailu_hbm, tailu_v)
    pltpu.sync_copy(tailm_hbm, tailm_v)
    pltpu.sync_copy(wtab_hbm, wv)

    semu = (semu0, semu1)
    semm = (semm0, semm1)
    urows = (urows0, urows1)
    mrows = (mrows0, mrows1)

    def start(j):
        sl = j % 2
        return (pltpu.async_copy(pu8.at[i8u[j]], urows[sl], semu[sl]),
                pltpu.async_copy(pm8.at[i8m[j]], mrows[sl], semm[sl]))

    lanes = lax.iota(jnp.int32, 16)
    inflight = {0: start(0), 1: start(1)}
    for j in range(NCHUNK):
        sl = j % 2
        cu, cm = inflight.pop(j)
        cu.wait()
        cm.wait()
        for g in range(8):
            gsl = pl.ds(g * 16, 16)
            offu = off_u[j, gsl]
            offm = off_m[j, gsl]
            tfu = tf_u[j, gsl]
            tfm = tf_m[j, gsl]
            inu = tfu >= 0
            inm = tfm >= 0
            tru = jnp.maximum(tfu, 0)
            trm = jnp.maximum(tfm, 0)
            rows = lanes + g * 16
            acc = wv[_B2_ROW, :]
            for jh in range(HID):
                vu = plsc.load_gather(urows[sl], [rows, offu + jh])
                vtu = plsc.load_gather(tailu_v, [tru, offu + jh])
                vu = jnp.where(inu, vtu, vu)
                vm = plsc.load_gather(mrows[sl], [rows, offm + jh])
                vtm = plsc.load_gather(tailm_v, [trm, offm + jh])
                vm = jnp.where(inm, vtm, vm)
                h = jnp.maximum(vu + vm + wv[_B1_ROW + jh, :], 0.0)
                acc = acc + h * wv[_W2_ROW + jh, :]
            outv[pl.ds(j * 128 + g * 16, 16)] = acc
        if j + 2 < NCHUNK:
            inflight[j + 2] = start(j + 2)

    plsc.subcore_barrier()
    pltpu.sync_copy(outv, out_hbm.at[pl.ds(wid * BPW, BPW)])


_sc_gather_mlp = functools.partial(
    pl.kernel,
    mesh=_sc_mesh,
    compiler_params=pltpu.CompilerParams(needs_layout_passes=False),
    out_type=jax.ShapeDtypeStruct((B,), jnp.float32),
    scratch_types=[
        pltpu.VMEM((128,), jnp.int32),
        pltpu.VMEM((128,), jnp.int32),
        pltpu.VMEM((128,), jnp.int32),
        pltpu.VMEM((128,), jnp.int32),
        pltpu.VMEM((128,), jnp.int32),
        pltpu.VMEM((128,), jnp.int32),
        pltpu.VMEM((128,), jnp.int32),
        pltpu.VMEM((128,), jnp.int32),
        pltpu.VMEM((NCHUNK, 128), jnp.int32),
        pltpu.VMEM((NCHUNK, 128), jnp.int32),
        pltpu.VMEM((NCHUNK, 128), jnp.int32),
        pltpu.VMEM((NCHUNK, 128), jnp.int32),
        pltpu.VMEM((TAIL_U * HIDP // 128, 128), jnp.float32),
        pltpu.VMEM((TAIL_M * HIDP // 128, 128), jnp.float32),
        pltpu.VMEM((128, 128), jnp.float32),
        pltpu.VMEM((128, 128), jnp.float32),
        pltpu.VMEM((128, 128), jnp.float32),
        pltpu.VMEM((128, 128), jnp.float32),
        pltpu.VMEM((_WTAB_ROWS, 16), jnp.float32),
        pltpu.VMEM((BPW,), jnp.float32),
        pltpu.SemaphoreType.DMA,
        pltpu.SemaphoreType.DMA,
        pltpu.SemaphoreType.DMA,
        pltpu.SemaphoreType.DMA,
    ],
)(_sc_body)


def kernel(users, movies, user_table, movie_table, W1, b1, W2, b2):
    w1aT = jnp.pad(W1[:EMB], ((0, 0), (0, HIDP - HID))).T
    w1bT = jnp.pad(W1[EMB:], ((0, 0), (0, HIDP - HID))).T
    tabTu = user_table.T
    tabTm = movie_table.T
    put = _project(tabTu, w1aT, N_U_MAIN, CHUNK_U, CHUNK_U)
    tailpu = _tail_proj(lax.slice(tabTu, (0, N_U_MAIN), (EMB, N_U)),
                        w1aT, TAIL_U)
    pmt = _project(tabTm, w1bT, N_M_MAIN, CHUNK_M, LAST_M)
    tailpm = _tail_proj(lax.slice(tabTm, (0, N_M_MAIN), (EMB, N_M)),
                        w1bT, TAIL_M)
    pu8 = put.T.reshape(N_U_MAIN * HIDP // 128, 128)
    pm8 = pmt.T.reshape(N_M_MAIN * HIDP // 128, 128)
    tail8u = tailpu.T.reshape(TAIL_U * HIDP // 128, 128)
    tail8m = tailpm.T.reshape(TAIL_M * HIDP // 128, 128)

    ones = jnp.ones((16,), jnp.float32)
    wtab = jnp.zeros((_WTAB_ROWS, 16), jnp.float32)
    wtab = wtab.at[_B1_ROW:_B1_ROW + HID].set(b1[:, None] * ones)
    wtab = wtab.at[_W2_ROW:_W2_ROW + HID].set(W2[:, 0][:, None] * ones)
    wtab = wtab.at[_B2_ROW].set(b2[0] * ones)

    ui = users.astype(jnp.int32)
    mi = movies.astype(jnp.int32)
    ui8 = ui >> 3
    mi8 = mi >> 3
    n_u_rows = N_U_MAIN // 8
    n_m_rows = N_M_MAIN // 8
    out = _sc_gather_mlp(
        pu8, pm8, tail8u, tail8m,
        jnp.minimum(ui8, n_u_rows - 1).reshape(NW, NCHUNK, 128),
        ((ui & 7) << 4).reshape(NW, NCHUNK, 128),
        jnp.where(ui8 >= n_u_rows, ui8 - n_u_rows, -1).reshape(
            NW, NCHUNK, 128),
        jnp.minimum(mi8, n_m_rows - 1).reshape(NW, NCHUNK, 128),
        ((mi & 7) << 4).reshape(NW, NCHUNK, 128),
        jnp.where(mi8 >= n_m_rows, mi8 - n_m_rows, -1).reshape(
            NW, NCHUNK, 128),
        wtab)
    return out.reshape(B, 1)


# speedup vs baseline: 2.9415x; 1.0595x over previous
import functools

import jax
import jax.numpy as jnp
from jax import lax
from jax.experimental import pallas as pl
from jax.experimental.pallas import tpu as pltpu
from jax.experimental.pallas import tpu_sc as plsc

B = 16384
EMB = 50
HID = 10
HIDP = 16
N_U = 1000000
N_M = 100000
CHUNK_U = 16128
NFULL_U = 62
N_U_MAIN = CHUNK_U * NFULL_U
TAIL_U = N_U - N_U_MAIN
CHUNK_M = 8192
LAST_M = 1664
N_M_MAIN = 12 * CHUNK_M + LAST_M
TAIL_M = N_M - N_M_MAIN

NC = 2
NS = 16
NW = NC * NS
BPW = B // NW
NCHUNK = BPW // 128

_B1_ROW = 0
_W2_ROW = 10
_B2_ROW = 20
_WTAB_ROWS = 24


def _proj_body(nchunks, chunk, last, tab_ref, w_ref, out_ref,
               xb, ob, semx, semo):
    i = pl.program_id(0)
    slot = i % 2
    ragged = last != chunk

    def in_copy(ci, sl, size):
        return pltpu.make_async_copy(
            tab_ref.at[:, pl.ds(ci * chunk, size)],
            xb.at[sl, :, pl.ds(0, size)], semx.at[sl])

    def out_copy(ci, sl, size):
        return pltpu.make_async_copy(
            ob.at[sl, :, pl.ds(0, size)],
            out_ref.at[:, pl.ds(ci * chunk, size)], semo.at[sl])

    def start_in(ci, sl):
        if ragged:
            @pl.when(ci == nchunks - 1)
            def _():
                in_copy(ci, sl, last).start()

            @pl.when(ci < nchunks - 1)
            def _():
                in_copy(ci, sl, chunk).start()
        else:
            in_copy(ci, sl, chunk).start()

    def wait_in(ci, sl):
        if ragged:
            @pl.when(ci == nchunks - 1)
            def _():
                in_copy(ci, sl, last).wait()

            @pl.when(ci < nchunks - 1)
            def _():
                in_copy(ci, sl, chunk).wait()
        else:
            in_copy(ci, sl, chunk).wait()

    def start_out(ci, sl):
        if ragged:
            @pl.when(ci == nchunks - 1)
            def _():
                out_copy(ci, sl, last).start()

            @pl.when(ci < nchunks - 1)
            def _():
                out_copy(ci, sl, chunk).start()
        else:
            out_copy(ci, sl, chunk).start()

    def wait_out(ci, sl):
        if ragged:
            @pl.when(ci == nchunks - 1)
            def _():
                out_copy(ci, sl, last).wait()

            @pl.when(ci < nchunks - 1)
            def _():
                out_copy(ci, sl, chunk).wait()
        else:
            out_copy(ci, sl, chunk).wait()

    @pl.when(i == 0)
    def _():
        start_in(0, 0)

    @pl.when((i + 1) < nchunks)
    def _():
        start_in(i + 1, 1 - slot)

    wait_in(i, slot)
    h = lax.dot_general(w_ref[...], xb[slot], (((1,), (0,)), ((), ())),
                        precision=lax.Precision.HIGHEST,
                        preferred_element_type=jnp.float32)
    @pl.when(i >= 2)
    def _():
        wait_out(i - 2, slot)

    ob[slot] = h
    start_out(i, slot)

    @pl.when(i == nchunks - 1)
    def _():
        wait_out(i, slot)

    @pl.when((i == nchunks - 1) & (i >= 1))
    def _():
        wait_out(i - 1, 1 - slot)


def _project(tabT, w16, n, chunk, last):
    nchunks = (n - last) // chunk + 1
    return pl.pallas_call(
        functools.partial(_proj_body, nchunks, chunk, last),
        grid=(nchunks,),
        in_specs=[
            pl.BlockSpec(memory_space=pl.ANY),
            pl.BlockSpec((HIDP, EMB), lambda i: (0, 0)),
        ],
        out_specs=pl.BlockSpec(memory_space=pl.ANY),
        out_shape=jax.ShapeDtypeStruct((HIDP, n), jnp.float32),
        scratch_shapes=[
            pltpu.VMEM((2, EMB, chunk), jnp.float32),
            pltpu.VMEM((2, HIDP, chunk), jnp.float32),
            pltpu.SemaphoreType.DMA((2,)),
            pltpu.SemaphoreType.DMA((2,)),
        ],
    )(tabT, w16)


def _tail_body(w_ref, t_ref, o_ref):
    o_ref[...] = lax.dot_general(w_ref[...], t_ref[...],
                                 (((1,), (0,)), ((), ())),
                                 precision=lax.Precision.HIGHEST,
                                 preferred_element_type=jnp.float32)


def _tail_proj(tailT, w16T, tail_n):
    return pl.pallas_call(
        _tail_body,
        out_shape=jax.ShapeDtypeStruct((HIDP, tail_n), jnp.float32),
    )(w16T, tailT)


_sc_mesh = plsc.VectorSubcoreMesh(core_axis_name="c", subcore_axis_name="s")


def _sc_body(pu8, pm8, tailu_hbm, tailm_hbm, uidx8_hbm, uoff_hbm, utf_hbm,
             midx8_hbm, moff_hbm, mtf_hbm, wtab_hbm,
             out_hbm, i8u0, i8u1, i8u2, i8u3, i8m0, i8m1, i8m2, i8m3,
             off_u, off_m, tf_u, tf_m, tailu_v, tailm_v,
             urows0, urows1, mrows0, mrows1,
             wv, outv, semu0, semu1, semm0, semm1):
    wid = lax.axis_index("s") * NC + lax.axis_index("c")
    i8u = (i8u0, i8u1, i8u2, i8u3)
    i8m = (i8m0, i8m1, i8m2, i8m3)
    for j in range(NCHUNK):
        pltpu.sync_copy(uidx8_hbm.at[wid, j], i8u[j])
        pltpu.sync_copy(midx8_hbm.at[wid, j], i8m[j])
    pltpu.sync_copy(uoff_hbm.at[wid], off_u)
    pltpu.sync_copy(moff_hbm.at[wid], off_m)
    pltpu.sync_copy(utf_hbm.at[wid], tf_u)
    pltpu.sync_copy(mtf_hbm.at[wid], tf_m)
    pltpu.sync_copy(tailu_hbm, tailu_v)
    pltpu.sync_copy(tailm_hbm, tailm_v)
    pltpu.sync_copy(wtab_hbm, wv)

    semu = (semu0, semu1)
    semm = (semm0, semm1)
    urows = (urows0, urows1)
    mrows = (mrows0, mrows1)

    def start(j):
        sl = j % 2
        return (pltpu.async_copy(pu8.at[i8u[j]], urows[sl], semu[sl]),
                pltpu.async_copy(pm8.at[i8m[j]], mrows[sl], semm[sl]))

    lanes = lax.iota(jnp.int32, 16)
    inflight = {0: start(0), 1: start(1)}
    for j in range(NCHUNK):
        sl = j % 2
        cu, cm = inflight.pop(j)
        cu.wait()
        cm.wait()
        for g in range(8):
            gsl = pl.ds(g * 16, 16)
            offu = off_u[j, gsl]
            offm = off_m[j, gsl]
            tfu = tf_u[j, gsl]
            tfm = tf_m[j, gsl]
            inu = tfu >= 0
            inm = tfm >= 0
            tru = jnp.maximum(tfu, 0)
            trm = jnp.maximum(tfm, 0)
            rows = lanes + g * 16
            acc = wv[_B2_ROW, :]
            for jh in range(HID):
                vu = plsc.load_gather(urows[sl], [rows, offu + jh])
                vtu = plsc.load_gather(tailu_v, [tru, offu + jh])
                vu = jnp.where(inu, vtu, vu)
                vm = plsc.load_gather(mrows[sl], [rows, offm + jh])
                vtm = plsc.load_gather(tailm_v, [trm, offm + jh])
                vm = jnp.where(inm, vtm, vm)
                h = jnp.maximum(vu + vm + wv[_B1_ROW + jh, :], 0.0)
                acc = acc + h * wv[_W2_ROW + jh, :]
            outv[pl.ds(j * 128 + g * 16, 16)] = acc
        if j + 2 < NCHUNK:
            inflight[j + 2] = start(j + 2)

    plsc.subcore_barrier()
    pltpu.sync_copy(outv, out_hbm.at[pl.ds(wid * BPW, BPW)])


_sc_gather_mlp = functools.partial(
    pl.kernel,
    mesh=_sc_mesh,
    compiler_params=pltpu.CompilerParams(needs_layout_passes=False),
    out_type=jax.ShapeDtypeStruct((B,), jnp.float32),
    scratch_types=[
        pltpu.VMEM((128,), jnp.int32),
        pltpu.VMEM((128,), jnp.int32),
        pltpu.VMEM((128,), jnp.int32),
        pltpu.VMEM((128,), jnp.int32),
        pltpu.VMEM((128,), jnp.int32),
        pltpu.VMEM((128,), jnp.int32),
        pltpu.VMEM((128,), jnp.int32),
        pltpu.VMEM((128,), jnp.int32),
        pltpu.VMEM((NCHUNK, 128), jnp.int32),
        pltpu.VMEM((NCHUNK, 128), jnp.int32),
        pltpu.VMEM((NCHUNK, 128), jnp.int32),
        pltpu.VMEM((NCHUNK, 128), jnp.int32),
        pltpu.VMEM((TAIL_U * HIDP // 128, 128), jnp.float32),
        pltpu.VMEM((TAIL_M * HIDP // 128, 128), jnp.float32),
        pltpu.VMEM((128, 128), jnp.float32),
        pltpu.VMEM((128, 128), jnp.float32),
        pltpu.VMEM((128, 128), jnp.float32),
        pltpu.VMEM((128, 128), jnp.float32),
        pltpu.VMEM((_WTAB_ROWS, 16), jnp.float32),
        pltpu.VMEM((BPW,), jnp.float32),
        pltpu.SemaphoreType.DMA,
        pltpu.SemaphoreType.DMA,
        pltpu.SemaphoreType.DMA,
        pltpu.SemaphoreType.DMA,
    ],
)(_sc_body)


def kernel(users, movies, user_table, movie_table, W1, b1, W2, b2):
    w1aT = jnp.pad(W1[:EMB], ((0, 0), (0, HIDP - HID))).T
    w1bT = jnp.pad(W1[EMB:], ((0, 0), (0, HIDP - HID))).T
    tabTu = user_table.T
    tabTm = movie_table.T
    put = _project(tabTu, w1aT, N_U_MAIN, CHUNK_U, CHUNK_U)
    tailpu = _tail_proj(lax.slice(tabTu, (0, N_U_MAIN), (EMB, N_U)),
                        w1aT, TAIL_U)
    pmt = _project(tabTm, w1bT, N_M_MAIN, CHUNK_M, LAST_M)
    tailpm = _tail_proj(lax.slice(tabTm, (0, N_M_MAIN), (EMB, N_M)),
                        w1bT, TAIL_M)
    pu8 = put.T.reshape(N_U_MAIN * HIDP // 128, 128)
    pm8 = pmt.T.reshape(N_M_MAIN * HIDP // 128, 128)
    tail8u = tailpu.T.reshape(TAIL_U * HIDP // 128, 128)
    tail8m = tailpm.T.reshape(TAIL_M * HIDP // 128, 128)

    ones = jnp.ones((16,), jnp.float32)
    wtab = jnp.zeros((_WTAB_ROWS, 16), jnp.float32)
    wtab = wtab.at[_B1_ROW:_B1_ROW + HID].set(b1[:, None] * ones)
    wtab = wtab.at[_W2_ROW:_W2_ROW + HID].set(W2[:, 0][:, None] * ones)
    wtab = wtab.at[_B2_ROW].set(b2[0] * ones)

    ui = users.astype(jnp.int32)
    mi = movies.astype(jnp.int32)
    ui8 = ui >> 3
    mi8 = mi >> 3
    n_u_rows = N_U_MAIN // 8
    n_m_rows = N_M_MAIN // 8
    out = _sc_gather_mlp(
        pu8, pm8, tail8u, tail8m,
        jnp.minimum(ui8, n_u_rows - 1).reshape(NW, NCHUNK, 128),
        ((ui & 7) << 4).reshape(NW, NCHUNK, 128),
        jnp.where(ui8 >= n_u_rows, ui8 - n_u_rows, -1).reshape(
            NW, NCHUNK, 128),
        jnp.minimum(mi8, n_m_rows - 1).reshape(NW, NCHUNK, 128),
        ((mi & 7) << 4).reshape(NW, NCHUNK, 128),
        jnp.where(mi8 >= n_m_rows, mi8 - n_m_rows, -1).reshape(
            NW, NCHUNK, 128),
        wtab)
    return out.reshape(B, 1)
